# Initial kernel scaffold; baseline (speedup 1.0000x reference)
import functools

import jax
import jax.numpy as jnp
from jax.experimental import pallas as pl
from jax.experimental.pallas import tpu as pltpu

N = 2048
E = 65536
C = 256
H = 4
HD = C // H
MAX_HOPS = 8

BR = 256
NRB = N // BR
INF_B = 30000.0


def _qkv_body(x_ref, wq_ref, wk_ref, wv_ref, bq_ref, bk_ref, bv_ref,
              q_ref, k_ref, v_ref):
    x = x_ref[...]
    hp = jax.lax.Precision.HIGHEST
    q_ref[...] = jnp.dot(x, wq_ref[...], precision=hp) + bq_ref[...]
    k_ref[...] = jnp.dot(x, wk_ref[...], precision=hp) + bk_ref[...]
    v_ref[...] = jnp.dot(x, wv_ref[...], precision=hp) + bv_ref[...]


def _qkv(x, Wq, Wk, Wv, bq, bk, bv):
    out = jax.ShapeDtypeStruct((N, C), jnp.float32)
    return pl.pallas_call(
        _qkv_body,
        out_shape=(out, out, out),
    )(x, Wq, Wk, Wv, bq.reshape(1, C), bk.reshape(1, C), bv.reshape(1, C))


def _bfs_body(acnt_ref, out_ref, abf_scr, reach_scr, dist_scr):
    k = pl.program_id(0)
    r = pl.program_id(1)
    rows = pl.ds(r * BR, BR)

    @pl.when(k == 0)
    def _init():
        a01 = acnt_ref[...] > 0.0
        abf_scr[rows, :] = a01.astype(jnp.bfloat16)
        reach_scr[rows, :] = a01.astype(jnp.bfloat16)
        ii = jax.lax.broadcasted_iota(jnp.int32, (BR, N), 0) + r * BR
        jj = jax.lax.broadcasted_iota(jnp.int32, (BR, N), 1)
        d = jnp.where(a01, 1.0, INF_B)
        d = jnp.where(ii == jj, 0.0, d)
        dist_scr[rows, :] = d.astype(jnp.bfloat16)

    @pl.when(k > 0)
    def _step():
        cnt = jnp.dot(reach_scr[rows, :], abf_scr[...],
                      preferred_element_type=jnp.float32)
        new = cnt > 0.0
        d = dist_scr[rows, :]
        hop = (k + 1).astype(jnp.float32).astype(jnp.bfloat16)
        dist_scr[rows, :] = jnp.where(new & (d > 1000.0), hop, d)
        reach_scr[rows, :] = new.astype(jnp.bfloat16)

    @pl.when(k == MAX_HOPS - 1)
    def _emit():
        d = dist_scr[rows, :].astype(jnp.float32)
        out_ref[...] = jnp.where(d > 1000.0, -1.0, d)


def _bfs_bias(Acnt):
    return pl.pallas_call(
        _bfs_body,
        grid=(MAX_HOPS, NRB),
        in_specs=[pl.BlockSpec((BR, N),
                               lambda k, r: (jnp.where(k == 0, r, 0), 0))],
        out_specs=pl.BlockSpec(
            (BR, N), lambda k, r: (jnp.where(k == MAX_HOPS - 1, r, 0), 0)),
        out_shape=jax.ShapeDtypeStruct((N, N), jnp.float32),
        scratch_shapes=[
            pltpu.VMEM((N, N), jnp.bfloat16),
            pltpu.VMEM((N, N), jnp.bfloat16),
            pltpu.VMEM((N, N), jnp.bfloat16),
        ],
    )(Acnt)


def _attn_body(q_ref, k_ref, v_ref, bias_ref, mc_ref, ac_ref, x_ref,
               eb_ref, g_ref, b_ref, y_ref):
    h = pl.program_id(1)
    hp = jax.lax.Precision.HIGHEST
    q = q_ref[...]
    kk = k_ref[...]
    s = jax.lax.dot_general(q, kk, (((1,), (1,)), ((), ())),
                            precision=hp) * (1.0 / (HD ** 0.5))
    s = s + bias_ref[...] + eb_ref[0, 0]
    mcnt = mc_ref[...]
    mask = mcnt > 0.0
    sm = jnp.where(mask, s, -1e30)
    m = jnp.max(sm, axis=1, keepdims=True)
    m = jnp.where(m < -1e29, 0.0, m)
    ex = mcnt * jnp.exp(sm - m)
    ssum = jnp.sum(ex, axis=1, keepdims=True)
    o = jnp.dot(ex, v_ref[...], precision=hp) / (ssum + 1e-16)
    y_ref[:, pl.ds(h * HD, HD)] = o

    @pl.when(h == H - 1)
    def _finalize():
        acc = y_ref[...]
        in_deg = jnp.sum(mcnt, axis=1, keepdims=True)
        out_deg = jnp.sum(ac_ref[...], axis=1, keepdims=True)
        hh = acc + x_ref[...] + (in_deg + out_deg)
        mu = jnp.mean(hh, axis=1, keepdims=True)
        var = jnp.mean((hh - mu) ** 2, axis=1, keepdims=True)
        y = (hh - mu) * jax.lax.rsqrt(var + 1e-5)
        y_ref[...] = y * g_ref[...] + b_ref[...]


def _attention(Q, K, V, bias, Mc, Ac, x, eb, gamma, beta):
    return pl.pallas_call(
        _attn_body,
        grid=(NRB, H),
        in_specs=[
            pl.BlockSpec((BR, HD), lambda r, h: (r, h)),
            pl.BlockSpec((N, HD), lambda r, h: (0, h)),
            pl.BlockSpec((N, HD), lambda r, h: (0, h)),
            pl.BlockSpec((BR, N), lambda r, h: (r, 0)),
            pl.BlockSpec((BR, N), lambda r, h: (r, 0)),
            pl.BlockSpec((BR, N), lambda r, h: (r, 0)),
            pl.BlockSpec((BR, C), lambda r, h: (r, 0)),
            pl.BlockSpec((1, 1), lambda r, h: (0, 0)),
            pl.BlockSpec((1, C), lambda r, h: (0, 0)),
            pl.BlockSpec((1, C), lambda r, h: (0, 0)),
        ],
        out_specs=pl.BlockSpec((BR, C), lambda r, h: (r, 0)),
        out_shape=jax.ShapeDtypeStruct((N, C), jnp.float32),
    )(Q, K, V, bias, Mc, Ac, x, eb.reshape(1, 1), gamma.reshape(1, C),
      beta.reshape(1, C))


def kernel(x, edge_index, Wq, bq, Wk, bk, Wv, bv, edge_bias, gamma, beta):
    src, dst = edge_index[0], edge_index[1]
    Mc = jnp.zeros((N, N), jnp.float32).at[dst, src].add(1.0)
    Ac = jnp.zeros((N, N), jnp.float32).at[src, dst].add(1.0)

    Q, K, V = _qkv(x, Wq, Wk, Wv, bq, bk, bv)
    bias = _bfs_bias(Ac)
    return _attention(Q, K, V, bias, Mc, Ac, x, edge_bias, gamma, beta)


# speedup vs baseline: 7.1226x; 7.1226x over previous
import functools

import jax
import jax.numpy as jnp
from jax.experimental import pallas as pl
from jax.experimental.pallas import tpu as pltpu

N = 2048
E = 65536
C = 256
H = 4
HD = C // H
MAX_HOPS = 8

BR = 256
NRB = N // BR
INF_B = 30000.0


def _qkv_body(x_ref, wq_ref, wk_ref, wv_ref, bq_ref, bk_ref, bv_ref,
              q_ref, k_ref, v_ref):
    x = x_ref[...]
    hp = jax.lax.Precision.HIGHEST
    q_ref[0] = jnp.dot(x, wq_ref[0], precision=hp) + bq_ref[0]
    k_ref[0] = jnp.dot(x, wk_ref[0], precision=hp) + bk_ref[0]
    v_ref[0] = jnp.dot(x, wv_ref[0], precision=hp) + bv_ref[0]


def _qkv(x, Wq, Wk, Wv, bq, bk, bv):
    out = jax.ShapeDtypeStruct((H, N, HD), jnp.float32)
    wspec = pl.BlockSpec((1, C, HD), lambda h: (h, 0, 0))
    bspec = pl.BlockSpec((1, 1, HD), lambda h: (h, 0, 0))
    ospec = pl.BlockSpec((1, N, HD), lambda h: (h, 0, 0))
    wh = lambda W: W.reshape(C, H, HD).transpose(1, 0, 2)
    bh = lambda b: b.reshape(1, H, HD).transpose(1, 0, 2)
    return pl.pallas_call(
        _qkv_body,
        grid=(H,),
        in_specs=[pl.BlockSpec((N, C), lambda h: (0, 0))] + [wspec] * 3
                 + [bspec] * 3,
        out_specs=(ospec, ospec, ospec),
        out_shape=(out, out, out),
    )(x, wh(Wq), wh(Wk), wh(Wv), bh(bq), bh(bk), bh(bv))


def _bfs_body(acnt_ref, out_ref, abf_scr, reach_scr, dist_scr):
    k = pl.program_id(0)
    r = pl.program_id(1)
    rows = pl.ds(r * BR, BR)

    @pl.when(k == 0)
    def _init():
        a01 = acnt_ref[...] > 0.0
        abf_scr[rows, :] = a01.astype(jnp.bfloat16)
        reach_scr[rows, :] = a01.astype(jnp.bfloat16)
        ii = jax.lax.broadcasted_iota(jnp.int32, (BR, N), 0) + r * BR
        jj = jax.lax.broadcasted_iota(jnp.int32, (BR, N), 1)
        d = jnp.where(a01, 1.0, INF_B)
        d = jnp.where(ii == jj, 0.0, d)
        dist_scr[rows, :] = d.astype(jnp.bfloat16)

    @pl.when(k > 0)
    def _step():
        cnt = jnp.dot(reach_scr[rows, :], abf_scr[...],
                      preferred_element_type=jnp.float32)
        new = cnt > 0.0
        d = dist_scr[rows, :]
        hop = (k + 1).astype(jnp.float32).astype(jnp.bfloat16)
        dist_scr[rows, :] = jnp.where(new & (d > 1000.0), hop, d)
        reach_scr[rows, :] = new.astype(jnp.bfloat16)

    @pl.when(k == MAX_HOPS - 1)
    def _emit():
        d = dist_scr[rows, :].astype(jnp.float32)
        out_ref[...] = jnp.where(d > 1000.0, -1.0, d)


def _bfs_bias(Acnt):
    return pl.pallas_call(
        _bfs_body,
        grid=(MAX_HOPS, NRB),
        in_specs=[pl.BlockSpec((BR, N),
                               lambda k, r: (jnp.where(k == 0, r, 0), 0))],
        out_specs=pl.BlockSpec(
            (BR, N), lambda k, r: (jnp.where(k == MAX_HOPS - 1, r, 0), 0)),
        out_shape=jax.ShapeDtypeStruct((N, N), jnp.float32),
        scratch_shapes=[
            pltpu.VMEM((N, N), jnp.bfloat16),
            pltpu.VMEM((N, N), jnp.bfloat16),
            pltpu.VMEM((N, N), jnp.bfloat16),
        ],
    )(Acnt)


def _attn_body(q_ref, k_ref, v_ref, bias_ref, mc_ref, ac_ref, x_ref,
               eb_ref, g_ref, b_ref, y_ref):
    hp = jax.lax.Precision.HIGHEST
    mcnt = mc_ref[...]
    mask = mcnt > 0.0
    base = bias_ref[...] + eb_ref[0, 0]
    outs = []
    for h in range(H):
        q = q_ref[h]
        s = jax.lax.dot_general(q, k_ref[h], (((1,), (1,)), ((), ())),
                                precision=hp) * (1.0 / (HD ** 0.5))
        s = s + base
        sm = jnp.where(mask, s, -1e30)
        m = jnp.max(sm, axis=1, keepdims=True)
        m = jnp.where(m < -1e29, 0.0, m)
        ex = mcnt * jnp.exp(sm - m)
        ssum = jnp.sum(ex, axis=1, keepdims=True)
        outs.append(jnp.dot(ex, v_ref[h], precision=hp) / (ssum + 1e-16))
    acc = jnp.concatenate(outs, axis=1)
    in_deg = jnp.sum(mcnt, axis=1, keepdims=True)
    out_deg = jnp.sum(ac_ref[...], axis=1, keepdims=True)
    hh = acc + x_ref[...] + (in_deg + out_deg)
    mu = jnp.mean(hh, axis=1, keepdims=True)
    var = jnp.mean((hh - mu) ** 2, axis=1, keepdims=True)
    y = (hh - mu) * jax.lax.rsqrt(var + 1e-5)
    y_ref[...] = y * g_ref[...] + b_ref[...]


def _attention(Q, K, V, bias, Mc, Ac, x, eb, gamma, beta):
    return pl.pallas_call(
        _attn_body,
        grid=(NRB,),
        in_specs=[
            pl.BlockSpec((H, BR, HD), lambda r: (0, r, 0)),
            pl.BlockSpec((H, N, HD), lambda r: (0, 0, 0)),
            pl.BlockSpec((H, N, HD), lambda r: (0, 0, 0)),
            pl.BlockSpec((BR, N), lambda r: (r, 0)),
            pl.BlockSpec((BR, N), lambda r: (r, 0)),
            pl.BlockSpec((BR, N), lambda r: (r, 0)),
            pl.BlockSpec((BR, C), lambda r: (r, 0)),
            pl.BlockSpec((1, 1), lambda r: (0, 0)),
            pl.BlockSpec((1, C), lambda r: (0, 0)),
            pl.BlockSpec((1, C), lambda r: (0, 0)),
        ],
        out_specs=pl.BlockSpec((BR, C), lambda r: (r, 0)),
        out_shape=jax.ShapeDtypeStruct((N, C), jnp.float32),
    )(Q, K, V, bias, Mc, Ac, x, eb.reshape(1, 1), gamma.reshape(1, C),
      beta.reshape(1, C))


def kernel(x, edge_index, Wq, bq, Wk, bk, Wv, bv, edge_bias, gamma, beta):
    src, dst = edge_index[0], edge_index[1]
    Mc = jnp.zeros((N, N), jnp.float32).at[dst, src].add(1.0)
    Ac = jnp.zeros((N, N), jnp.float32).at[src, dst].add(1.0)

    Q, K, V = _qkv(x, Wq, Wk, Wv, bq, bk, bv)
    bias = _bfs_bias(Ac)
    return _attention(Q, K, V, bias, Mc, Ac, x, edge_bias, gamma, beta)
